# baseline (speedup 1.0000x reference)
import functools

import jax
import jax.numpy as jnp
from jax import lax
from jax.experimental import pallas as pl
from jax.experimental.pallas import tpu as pltpu
from jax.experimental.pallas import tpu_sc as plsc

BATCH = 4096
SEQ = 200
EMBED_DIM = 64
WORDS = EMBED_DIM // 2

NUM_CORES = 2
NUM_SUBCORES = 16
NW = NUM_CORES * NUM_SUBCORES
B_PER_W = BATCH // NW
IDX_PER_W = B_PER_W * SEQ
GROUP = 2
GROUP_ROWS = GROUP * SEQ
NREG = WORDS // 16
UNROLL = 8

TBLOCK = 4096
NSEG = 4
NBLK = 62
SEG_ROWS = NBLK * TBLOCK


def _pool_body(x_hbm, emb_hbm, out_hbm, idx_v, rows0_v, rows1_v, stage_v, sem0, sem1):
    wid = lax.axis_index("s") * NUM_CORES + lax.axis_index("c")
    pltpu.sync_copy(x_hbm.at[pl.ds(wid * IDX_PER_W, IDX_PER_W)], idx_v)

    bufs = ((rows0_v, sem0), (rows1_v, sem1))
    himask = jnp.full((16,), -65536, jnp.int32)

    def issue(g, rv, sem):
        pltpu.async_copy(
            emb_hbm.at[idx_v.at[pl.ds(g * GROUP_ROWS, GROUP_ROWS)]],
            rv,
            sem,
        )

    def wait(rv, sem):
        pltpu.make_async_copy(emb_hbm.at[pl.ds(0, GROUP_ROWS), :], rv, sem).wait()

    def accumulate(g, rv):
        def acc_body(i, acc):
            acc = list(acc)
            for u in range(UNROLL):
                r = i * UNROLL + u
                for k in range(GROUP):
                    for c in range(NREG):
                        w = rv[k * SEQ + r, pl.ds(c * 16, 16)]
                        lo = plsc.bitcast(lax.shift_left(w, 16), jnp.float32)
                        hi = plsc.bitcast(lax.bitwise_and(w, himask), jnp.float32)
                        j = k * 2 * NREG
                        acc[j + c] = acc[j + c] + lo
                        acc[j + NREG + c] = acc[j + NREG + c] + hi
            return tuple(acc)

        zeros = tuple(jnp.zeros((16,), jnp.float32) for _ in range(GROUP * 2 * NREG))
        acc = lax.fori_loop(0, SEQ // UNROLL, acc_body, zeros)
        for k in range(GROUP):
            for c in range(NREG):
                j = k * 2 * NREG
                stage_v[g * GROUP + k, pl.ds(c * 16, 16)] = acc[j + c]
                stage_v[g * GROUP + k, pl.ds(32 + c * 16, 16)] = acc[j + NREG + c]

    n_groups = B_PER_W // GROUP
    issue(0, rows0_v, sem0)
    issue(1, rows1_v, sem1)

    def body(t, carry):
        for b, (rv, sem) in enumerate(bufs):
            g = 2 * t + b
            wait(rv, sem)
            accumulate(g, rv)
            issue(g + 2, rv, sem)
        return carry

    lax.fori_loop(0, n_groups // 2 - 1, body, 0)
    for b, (rv, sem) in enumerate(bufs):
        g = n_groups - 2 + b
        wait(rv, sem)
        accumulate(g, rv)

    pltpu.sync_copy(stage_v, out_hbm.at[pl.ds(wid * B_PER_W, B_PER_W), :])


@jax.jit
def _pool(x_flat, emb_words):
    mesh = plsc.VectorSubcoreMesh(
        core_axis_name="c",
        subcore_axis_name="s",
        num_cores=NUM_CORES,
        num_subcores=NUM_SUBCORES,
    )
    return pl.kernel(
        _pool_body,
        out_type=jax.ShapeDtypeStruct((BATCH, EMBED_DIM), jnp.float32),
        mesh=mesh,
        scratch_types=[
            pltpu.VMEM((IDX_PER_W,), jnp.int32),
            pltpu.VMEM((GROUP_ROWS, WORDS), jnp.int32),
            pltpu.VMEM((GROUP_ROWS, WORDS), jnp.int32),
            pltpu.VMEM((B_PER_W, EMBED_DIM), jnp.float32),
            pltpu.SemaphoreType.DMA,
            pltpu.SemaphoreType.DMA,
        ],
        compiler_params=pltpu.CompilerParams(
            use_tc_tiling_on_sc=False, needs_layout_passes=False
        ),
    )(x_flat, emb_words)


def _pack_words(tt, eye):
    t = lax.dot_general(
        tt, eye, (((0,), (0,)), ((), ())), preferred_element_type=jnp.float32
    )
    rb = t.astype(jnp.bfloat16).astype(jnp.float32)
    bits = lax.bitcast_convert_type(rb, jnp.int32)
    lo = lax.shift_right_logical(bits[:, :WORDS], 16)
    hi = lax.bitwise_and(bits[:, WORDS:], jnp.int32(-65536))
    return lax.bitwise_or(hi, lo)


def _tpack_body(s0_ref, s1_ref, s2_ref, s3_ref, eye_ref, o_ref):
    eye = eye_ref[...]
    parts = []
    for ref in (s0_ref, s1_ref, s2_ref, s3_ref):
        parts.append(_pack_words(ref[...], eye))
    o_ref[...] = jnp.concatenate(parts, axis=1)


def _transpose_pack(embt):
    max_blk = (embt.shape[1] - 1) // TBLOCK
    specs = []
    for s in range(NSEG):
        specs.append(
            pl.BlockSpec(
                (EMBED_DIM, TBLOCK),
                functools.partial(
                    lambda i, off: (0, jnp.minimum(i + off, max_blk)),
                    off=s * NBLK,
                ),
            )
        )
    specs.append(pl.BlockSpec((EMBED_DIM, EMBED_DIM), lambda i: (0, 0)))
    return pl.pallas_call(
        _tpack_body,
        grid=(NBLK,),
        in_specs=specs,
        out_specs=pl.BlockSpec((TBLOCK, NSEG * WORDS), lambda i: (i, 0)),
        out_shape=jax.ShapeDtypeStruct((SEG_ROWS, NSEG * WORDS), jnp.int32),
    )(embt, embt, embt, embt, jnp.eye(EMBED_DIM, dtype=jnp.float32))


def _mlp_body(h_ref, w1_ref, b1_ref, w2_ref, b2_ref, o_ref):
    h = h_ref[...] * (1.0 / SEQ)
    z = jnp.dot(h, w1_ref[...], preferred_element_type=jnp.float32) + b1_ref[...]
    z = jnp.maximum(z, 0.0)
    o_ref[...] = jnp.dot(z, w2_ref[...], preferred_element_type=jnp.float32) + b2_ref[...]


@jax.jit
def _mlp(pooled, W1, b1, W2, b2):
    return pl.pallas_call(
        _mlp_body,
        out_shape=jax.ShapeDtypeStruct((BATCH, 1), jnp.float32),
    )(pooled, W1, b1.reshape(1, 32), W2, b2.reshape(1, 1))


def kernel(x, emb, W1, b1, W2, b2):
    xf = x.reshape(BATCH * SEQ).astype(jnp.int32)
    s = xf // SEG_ROWS
    u = xf - s * SEG_ROWS
    x_flat = 4 * u + s
    emb_pack = _transpose_pack(emb.T)
    emb_words = emb_pack.reshape(NSEG * SEG_ROWS, WORDS)
    pooled = _pool(x_flat, emb_words)
    return _mlp(pooled, W1, b1, W2, b2)


# speedup vs baseline: 1.3823x; 1.3823x over previous
import functools

import jax
import jax.numpy as jnp
from jax import lax
from jax.experimental import pallas as pl
from jax.experimental.pallas import tpu as pltpu
from jax.experimental.pallas import tpu_sc as plsc

BATCH = 4096
SEQ = 200
EMBED_DIM = 64
WORDS = EMBED_DIM // 2

NUM_CORES = 2
NUM_SUBCORES = 16
NW = NUM_CORES * NUM_SUBCORES
B_PER_W = BATCH // NW
IDX_PER_W = B_PER_W * SEQ
GROUP = 2
GROUP_ROWS = GROUP * SEQ
NREG = WORDS // 16
UNROLL = 8

TBLOCK = 4096
NSEG = 4
NBLK = 62
SEG_ROWS = NBLK * TBLOCK


def _pool_body(x_hbm, emb_hbm, out_hbm, idx_v, rows0_v, rows1_v, stage_v, sem0, sem1):
    wid = lax.axis_index("s") * NUM_CORES + lax.axis_index("c")
    pltpu.sync_copy(x_hbm.at[pl.ds(wid * IDX_PER_W, IDX_PER_W)], idx_v)

    bufs = ((rows0_v, sem0), (rows1_v, sem1))
    himask = jnp.full((16,), -65536, jnp.int32)

    def issue(g, rv, sem):
        pltpu.async_copy(
            emb_hbm.at[idx_v.at[pl.ds(g * GROUP_ROWS, GROUP_ROWS)]],
            rv,
            sem,
        )

    def wait(rv, sem):
        pltpu.make_async_copy(emb_hbm.at[pl.ds(0, GROUP_ROWS), :], rv, sem).wait()

    def accumulate(g, rv):
        def acc_body(i, acc):
            acc = list(acc)
            for u in range(UNROLL):
                r = i * UNROLL + u
                for k in range(GROUP):
                    for c in range(NREG):
                        w = rv[k * SEQ + r, pl.ds(c * 16, 16)]
                        lo = plsc.bitcast(lax.shift_left(w, 16), jnp.float32)
                        hi = plsc.bitcast(lax.bitwise_and(w, himask), jnp.float32)
                        j = k * 2 * NREG
                        acc[j + c] = acc[j + c] + lo
                        acc[j + NREG + c] = acc[j + NREG + c] + hi
            return tuple(acc)

        zeros = tuple(jnp.zeros((16,), jnp.float32) for _ in range(GROUP * 2 * NREG))
        acc = lax.fori_loop(0, SEQ // UNROLL, acc_body, zeros)
        for k in range(GROUP):
            for c in range(NREG):
                j = k * 2 * NREG
                stage_v[g * GROUP + k, pl.ds(c * 16, 16)] = acc[j + c]
                stage_v[g * GROUP + k, pl.ds(32 + c * 16, 16)] = acc[j + NREG + c]

    n_groups = B_PER_W // GROUP
    issue(0, rows0_v, sem0)
    issue(1, rows1_v, sem1)

    def body(t, carry):
        for b, (rv, sem) in enumerate(bufs):
            g = 2 * t + b
            wait(rv, sem)
            accumulate(g, rv)
            issue(g + 2, rv, sem)
        return carry

    lax.fori_loop(0, n_groups // 2 - 1, body, 0)
    for b, (rv, sem) in enumerate(bufs):
        g = n_groups - 2 + b
        wait(rv, sem)
        accumulate(g, rv)

    pltpu.sync_copy(stage_v, out_hbm.at[pl.ds(wid * B_PER_W, B_PER_W), :])


@jax.jit
def _pool(x_flat, emb_words):
    mesh = plsc.VectorSubcoreMesh(
        core_axis_name="c",
        subcore_axis_name="s",
        num_cores=NUM_CORES,
        num_subcores=NUM_SUBCORES,
    )
    return pl.kernel(
        _pool_body,
        out_type=jax.ShapeDtypeStruct((BATCH, EMBED_DIM), jnp.float32),
        mesh=mesh,
        scratch_types=[
            pltpu.VMEM((IDX_PER_W,), jnp.int32),
            pltpu.VMEM((GROUP_ROWS, WORDS), jnp.int32),
            pltpu.VMEM((GROUP_ROWS, WORDS), jnp.int32),
            pltpu.VMEM((B_PER_W, EMBED_DIM), jnp.float32),
            pltpu.SemaphoreType.DMA,
            pltpu.SemaphoreType.DMA,
        ],
        compiler_params=pltpu.CompilerParams(
            use_tc_tiling_on_sc=False, needs_layout_passes=False
        ),
    )(x_flat, emb_words)


def _pack_words(tt):
    rb = tt.astype(jnp.bfloat16).astype(jnp.float32)
    bits = lax.bitcast_convert_type(rb, jnp.int32)
    lo = lax.shift_right_logical(bits[:WORDS, :], 16)
    hi = lax.bitwise_and(bits[WORDS:, :], jnp.int32(-65536))
    return jnp.transpose(lax.bitwise_or(hi, lo), (1, 0))


def _tpack_body(s0_ref, s1_ref, s2_ref, s3_ref, o_ref):
    parts = []
    for ref in (s0_ref, s1_ref, s2_ref, s3_ref):
        parts.append(_pack_words(ref[...]))
    o_ref[...] = jnp.concatenate(parts, axis=1)


def _transpose_pack(embt):
    max_blk = (embt.shape[1] - 1) // TBLOCK
    specs = []
    for s in range(NSEG):
        specs.append(
            pl.BlockSpec(
                (EMBED_DIM, TBLOCK),
                functools.partial(
                    lambda i, off: (0, jnp.minimum(i + off, max_blk)),
                    off=s * NBLK,
                ),
            )
        )
    return pl.pallas_call(
        _tpack_body,
        grid=(NBLK,),
        in_specs=specs,
        out_specs=pl.BlockSpec((TBLOCK, NSEG * WORDS), lambda i: (i, 0)),
        out_shape=jax.ShapeDtypeStruct((SEG_ROWS, NSEG * WORDS), jnp.int32),
    )(embt, embt, embt, embt)


def _mlp_body(h_ref, w1_ref, b1_ref, w2_ref, b2_ref, o_ref):
    h = h_ref[...] * (1.0 / SEQ)
    z = jnp.dot(h, w1_ref[...], preferred_element_type=jnp.float32) + b1_ref[...]
    z = jnp.maximum(z, 0.0)
    o_ref[...] = jnp.dot(z, w2_ref[...], preferred_element_type=jnp.float32) + b2_ref[...]


@jax.jit
def _mlp(pooled, W1, b1, W2, b2):
    return pl.pallas_call(
        _mlp_body,
        out_shape=jax.ShapeDtypeStruct((BATCH, 1), jnp.float32),
    )(pooled, W1, b1.reshape(1, 32), W2, b2.reshape(1, 1))


def kernel(x, emb, W1, b1, W2, b2):
    xf = x.reshape(BATCH * SEQ).astype(jnp.int32)
    s = xf // SEG_ROWS
    u = xf - s * SEG_ROWS
    x_flat = 4 * u + s
    emb_pack = _transpose_pack(emb.T)
    emb_words = emb_pack.reshape(NSEG * SEG_ROWS, WORDS)
    pooled = _pool(x_flat, emb_words)
    return _mlp(pooled, W1, b1, W2, b2)
